# Initial kernel scaffold; baseline (speedup 1.0000x reference)
import jax
import jax.numpy as jnp
from jax.experimental import pallas as pl


def kernel(cfs, cmps):
    raise NotImplementedError("write your pallas kernel here")


# speedup vs baseline: 1.4067x; 1.4067x over previous
import jax
import jax.numpy as jnp
from jax.experimental import pallas as pl
from jax.experimental.pallas import tpu as pltpu


def _cdiv(a, b):
    return (a + b - 1) // b


def _mm_kernel(a_ref, b_ref, o_ref):
    a = a_ref[...].astype(jnp.bfloat16)
    b = b_ref[...].astype(jnp.bfloat16)
    o_ref[...] = jnp.dot(a, b, preferred_element_type=jnp.float32).astype(
        o_ref.dtype
    )


def _mm_1d(cfs, cmps, tn):
    M, K = cfs.shape
    _, N = cmps.shape
    grid_n = _cdiv(N, tn)
    cost = pl.CostEstimate(
        flops=2 * M * N * K,
        transcendentals=0,
        bytes_accessed=4 * (M * K + K * N + M * N),
    )
    return pl.pallas_call(
        _mm_kernel,
        out_shape=jax.ShapeDtypeStruct((M, N), cfs.dtype),
        grid=(grid_n,),
        in_specs=[
            pl.BlockSpec((M, K), lambda j: (0, 0)),
            pl.BlockSpec((K, tn), lambda j: (0, j)),
        ],
        out_specs=pl.BlockSpec((M, tn), lambda j: (0, j)),
        compiler_params=pltpu.CompilerParams(
            dimension_semantics=("parallel",),
        ),
        cost_estimate=cost,
    )(cfs, cmps)


def _mm_2d(cfs, cmps, tm, tn):
    M, K = cfs.shape
    _, N = cmps.shape
    cost = pl.CostEstimate(
        flops=2 * M * N * K,
        transcendentals=0,
        bytes_accessed=4 * (M * K + K * N * _cdiv(M, tm) + M * N),
    )
    return pl.pallas_call(
        _mm_kernel,
        out_shape=jax.ShapeDtypeStruct((M, N), cfs.dtype),
        grid=(_cdiv(M, tm), _cdiv(N, tn)),
        in_specs=[
            pl.BlockSpec((tm, K), lambda i, j: (i, 0)),
            pl.BlockSpec((K, tn), lambda i, j: (0, j)),
        ],
        out_specs=pl.BlockSpec((tm, tn), lambda i, j: (i, j)),
        compiler_params=pltpu.CompilerParams(
            dimension_semantics=("parallel", "parallel"),
        ),
        cost_estimate=cost,
    )(cfs, cmps)


def kernel(cfs, cmps):
    M, K = cfs.shape
    K2, N = cmps.shape
    assert K == K2, "inner dimensions must match"

    tn = 1024
    while tn > 128 and _cdiv(N, tn) < 2:
        tn //= 2

    a_bytes = 4 * M * K
    stream_bytes = 2 * (4 * K * tn + 4 * M * tn)
    if a_bytes + stream_bytes <= 40 * 1024 * 1024:
        return _mm_1d(cfs, cmps, tn)
    tm = 512
    return _mm_2d(cfs, cmps, tm, tn)
